# baseline (speedup 1.0000x reference)
import functools

import jax
import jax.numpy as jnp
from jax import lax
from jax.experimental import pallas as pl
from jax.experimental.pallas import tpu as pltpu
from jax.experimental.pallas import tpu_sc as plsc

N = 10000
E = 320000
D = 128
G = 64
OUT = 10

NC = 2
NS = 16
NW = NC * NS
CHUNK = 128
NCHUNK = 80
EPW = CHUNK * NCHUNK
EPAD = NW * EPW
NPAD = 10112
RPT = NPAD // NS

NPHASE = 2
CPP = NCHUNK // NPHASE

RB = 1000
GRID = N // RB


def _prep_body(x_ref, deg_ref, w_ref, y_ref, dinv_ref):
    deg = deg_ref[0, :, 0:1] + deg_ref[1, :, 0:1] + 1.0
    dinv = lax.rsqrt(deg)
    xw = jnp.dot(x_ref[...], w_ref[...], preferred_element_type=jnp.float32)
    dinv_ref[...] = dinv
    y_ref[...] = dinv * xw


def _tc_prep(x, deg_p, w1):
    return pl.pallas_call(
        _prep_body,
        grid=(GRID,),
        in_specs=[
            pl.BlockSpec((RB, D), lambda i: (i, 0)),
            pl.BlockSpec((2, RB, D), lambda i: (0, i, 0)),
            pl.BlockSpec((D, D), lambda i: (0, 0)),
        ],
        out_specs=[
            pl.BlockSpec((RB, D), lambda i: (i, 0)),
            pl.BlockSpec((RB, 1), lambda i: (i, 0)),
        ],
        out_shape=[
            jax.ShapeDtypeStruct((N, D), jnp.float32),
            jax.ShapeDtypeStruct((N, 1), jnp.float32),
        ],
    )(x, deg_p, w1)


def _combine_body(p_ref, dinv_ref, b_ref, w_ref, ynext_ref):
    agg = p_ref[0] + p_ref[1]
    h = jnp.maximum(dinv_ref[...] * agg + b_ref[...], 0.0)
    ynext_ref[...] = dinv_ref[...] * jnp.dot(
        h, w_ref[...], preferred_element_type=jnp.float32)


def _tc_combine(p, dinv, b, wn):
    return pl.pallas_call(
        _combine_body,
        grid=(GRID,),
        in_specs=[
            pl.BlockSpec((2, RB, D), lambda i: (0, i, 0)),
            pl.BlockSpec((RB, 1), lambda i: (i, 0)),
            pl.BlockSpec((1, D), lambda i: (0, 0)),
            pl.BlockSpec((D, D), lambda i: (0, 0)),
        ],
        out_specs=pl.BlockSpec((RB, D), lambda i: (i, 0)),
        out_shape=jax.ShapeDtypeStruct((N, D), jnp.float32),
    )(p, dinv, b, wn)


def _readout_body(p_ref, dinv_ref, b_ref, batch_ref, wm_ref, bm_ref,
                  emb_ref, gemb_ref, logits_ref, probs_ref,
                  gacc, cacc):
    i = pl.program_id(0)
    agg = p_ref[0] + p_ref[1]
    h = jnp.maximum(dinv_ref[...] * agg + b_ref[...], 0.0)
    emb_ref[...] = h

    onehot = (batch_ref[0] == lax.broadcasted_iota(jnp.int32, (G, RB), 0))
    onehot = onehot.astype(jnp.float32)
    gpart = jnp.dot(onehot, h, preferred_element_type=jnp.float32)
    cpart = jnp.broadcast_to(jnp.sum(onehot, axis=1, keepdims=True), (G, D))

    @pl.when(i == 0)
    def _():
        gacc[...] = gpart
        cacc[...] = cpart

    @pl.when(i > 0)
    def _():
        gacc[...] = gacc[...] + gpart
        cacc[...] = cacc[...] + cpart

    @pl.when(i == GRID - 1)
    def _():
        ge = gacc[...] / jnp.maximum(cacc[...], 1.0)
        gemb_ref[...] = ge
        logits = jnp.dot(ge, wm_ref[...], preferred_element_type=jnp.float32)
        logits = logits + bm_ref[...]
        logits_ref[...] = logits
        m = jnp.max(logits, axis=-1, keepdims=True)
        ex = jnp.exp(logits - m)
        probs_ref[...] = ex / jnp.sum(ex, axis=-1, keepdims=True)


def _tc_readout(p, dinv, b, batch2d, wm, bm):
    return pl.pallas_call(
        _readout_body,
        grid=(GRID,),
        in_specs=[
            pl.BlockSpec((2, RB, D), lambda i: (0, i, 0)),
            pl.BlockSpec((RB, 1), lambda i: (i, 0)),
            pl.BlockSpec((1, D), lambda i: (0, 0)),
            pl.BlockSpec((1, 1, RB), lambda i: (i, 0, 0)),
            pl.BlockSpec((D, OUT), lambda i: (0, 0)),
            pl.BlockSpec((1, OUT), lambda i: (0, 0)),
        ],
        out_specs=[
            pl.BlockSpec((RB, D), lambda i: (i, 0)),
            pl.BlockSpec((G, D), lambda i: (0, 0)),
            pl.BlockSpec((G, OUT), lambda i: (0, 0)),
            pl.BlockSpec((G, OUT), lambda i: (0, 0)),
        ],
        out_shape=[
            jax.ShapeDtypeStruct((N, D), jnp.float32),
            jax.ShapeDtypeStruct((G, D), jnp.float32),
            jax.ShapeDtypeStruct((G, OUT), jnp.float32),
            jax.ShapeDtypeStruct((G, OUT), jnp.float32),
        ],
        scratch_shapes=[
            pltpu.VMEM((G, D), jnp.float32),
            pltpu.VMEM((G, D), jnp.float32),
        ],
    )(p, dinv, b, batch2d, wm, bm)


@functools.lru_cache(maxsize=None)
def _sc_mesh():
    return plsc.VectorSubcoreMesh(
        core_axis_name="c", subcore_axis_name="s",
        num_cores=NC, num_subcores=NS)


def _zero_rows(ref, nrows):
    def zb(i, _):
        for k in range(D // 16):
            ref[i, pl.ds(k * 16, 16)] = jnp.zeros((16,), jnp.float32)
        return 0
    lax.fori_loop(0, nrows, zb, 0)


def _fill_const(ref, nrows, val):
    def fb(i, _):
        for k in range(D // 16):
            ref[i, pl.ds(k * 16, 16)] = jnp.full((16,), val, jnp.float32)
        return 0
    lax.fori_loop(0, nrows, fb, 0)


def _deg_body(dst_hbm, out_hbm, dstv, ones_v, acc):
    cid = lax.axis_index("c")
    sid = lax.axis_index("s")
    wid = cid * NS + sid

    pltpu.sync_copy(dst_hbm.at[wid], dstv)

    _fill_const(ones_v, CHUNK, 0.0)
    base = sid * RPT
    for t in range(RPT // CHUNK):
        pltpu.sync_copy(ones_v, acc.at[pl.ds(base + t * CHUNK, CHUNK)])
    rem = RPT % CHUNK
    pltpu.sync_copy(ones_v.at[pl.ds(0, rem)],
                    acc.at[pl.ds(base + (RPT // CHUNK) * CHUNK, rem)])

    _fill_const(ones_v, CHUNK, 1.0)
    plsc.subcore_barrier()

    def body(j, _):
        pltpu.sync_copy(ones_v, acc.at[dstv.at[j]], add=True)
        return 0
    lax.fori_loop(0, NCHUNK, body, 0)

    plsc.subcore_barrier()
    for t in range(RPT // CHUNK):
        pltpu.sync_copy(acc.at[pl.ds(base + t * CHUNK, CHUNK)], ones_v)
        pltpu.sync_copy(ones_v, out_hbm.at[cid, pl.ds(base + t * CHUNK, CHUNK)])
    rb = base + (RPT // CHUNK) * CHUNK
    pltpu.sync_copy(acc.at[pl.ds(rb, rem)], ones_v.at[pl.ds(0, rem)])
    pltpu.sync_copy(ones_v.at[pl.ds(0, rem)],
                    out_hbm.at[cid, pl.ds(rb, rem)])


@functools.lru_cache(maxsize=None)
def _sc_deg_kernel():
    return pl.kernel(
        _deg_body,
        out_type=jax.ShapeDtypeStruct((NC, NPAD, D), jnp.float32),
        mesh=_sc_mesh(),
        scratch_types=[
            pltpu.VMEM((NCHUNK, CHUNK), jnp.int32),
            pltpu.VMEM((CHUNK, D), jnp.float32),
            pltpu.VMEM_SHARED((NPAD, D), jnp.float32),
        ],
    )


def _sc_deg(dst_p):
    return _sc_deg_kernel()(dst_p)


def _scat_body(y_hbm, src_hbm, dst_hbm, out_hbm,
               srcv, dstv, rows0, rows1, acc, sem0, sem1, sems0, sems1):
    cid = lax.axis_index("c")
    sid = lax.axis_index("s")
    wid = cid * NS + sid

    @pl.when(cid == 0)
    def _():
        base = sid * RPT
        for t in range(4):
            pltpu.sync_copy(y_hbm.at[pl.ds(base + t * CHUNK, CHUNK)], rows0)
            pltpu.sync_copy(rows0, acc.at[pl.ds(base + t * CHUNK, CHUNK)])

        @pl.when(sid < NS - 1)
        def _():
            pltpu.sync_copy(y_hbm.at[pl.ds(base + 4 * CHUNK, RPT - 4 * CHUNK)],
                            rows0.at[pl.ds(0, RPT - 4 * CHUNK)])
            pltpu.sync_copy(rows0.at[pl.ds(0, RPT - 4 * CHUNK)],
                            acc.at[pl.ds(base + 4 * CHUNK, RPT - 4 * CHUNK)])

        @pl.when(sid == NS - 1)
        def _():
            b2_ = (NS - 1) * RPT + 4 * CHUNK
            pltpu.sync_copy(y_hbm.at[pl.ds(b2_, N - b2_)],
                            rows0.at[pl.ds(0, N - b2_)])
            pltpu.sync_copy(rows0.at[pl.ds(0, N - b2_)],
                            acc.at[pl.ds(b2_, N - b2_)])
            _zero_rows(rows0, NPAD - N)
            pltpu.sync_copy(rows0.at[pl.ds(0, NPAD - N)],
                            acc.at[pl.ds(N, NPAD - N)])

    @pl.when(cid == 1)
    def _():
        _zero_rows(rows0, CHUNK)
        base = sid * RPT
        for t in range(RPT // CHUNK):
            pltpu.sync_copy(rows0, acc.at[pl.ds(base + t * CHUNK, CHUNK)])
        rem = RPT % CHUNK
        pltpu.sync_copy(rows0.at[pl.ds(0, rem)],
                        acc.at[pl.ds(base + (RPT // CHUNK) * CHUNK, rem)])

    plsc.subcore_barrier()

    for ph in range(NPHASE):
        pltpu.sync_copy(src_hbm.at[wid, pl.ds(ph * CPP, CPP)], srcv)
        pltpu.sync_copy(dst_hbm.at[wid, pl.ds(ph * CPP, CPP)], dstv)

        pltpu.async_copy(y_hbm.at[srcv.at[0]], rows0, sem0)
        pltpu.async_copy(y_hbm.at[srcv.at[1]], rows1, sem1)

        def body(jj, _):
            j0 = 2 * jj
            pltpu.make_async_copy(y_hbm.at[srcv.at[j0]], rows0, sem0).wait()
            pltpu.async_copy(rows0, acc.at[dstv.at[j0]], sems0, add=True)

            pltpu.make_async_copy(y_hbm.at[srcv.at[j0 + 1]], rows1, sem1).wait()
            pltpu.async_copy(rows1, acc.at[dstv.at[j0 + 1]], sems1, add=True)

            pltpu.make_async_copy(rows0, acc.at[dstv.at[j0]], sems0).wait()

            @pl.when(jj < CPP // 2 - 1)
            def _():
                pltpu.async_copy(y_hbm.at[srcv.at[j0 + 2]], rows0, sem0)

            pltpu.make_async_copy(rows1, acc.at[dstv.at[j0 + 1]], sems1).wait()

            @pl.when(jj < CPP // 2 - 1)
            def _():
                pltpu.async_copy(y_hbm.at[srcv.at[j0 + 3]], rows1, sem1)
            return 0

        lax.fori_loop(0, CPP // 2, body, 0)

    plsc.subcore_barrier()
    base = sid * RPT
    for t in range(RPT // CHUNK):
        pltpu.sync_copy(acc.at[pl.ds(base + t * CHUNK, CHUNK)], rows0)
        pltpu.sync_copy(rows0, out_hbm.at[cid, pl.ds(base + t * CHUNK, CHUNK)])
    rem = RPT % CHUNK
    rb = base + (RPT // CHUNK) * CHUNK
    pltpu.sync_copy(acc.at[pl.ds(rb, rem)], rows0.at[pl.ds(0, rem)])
    pltpu.sync_copy(rows0.at[pl.ds(0, rem)],
                    out_hbm.at[cid, pl.ds(rb, rem)])


@functools.lru_cache(maxsize=None)
def _sc_scatter_kernel():
    return pl.kernel(
        _scat_body,
        out_type=jax.ShapeDtypeStruct((NC, NPAD, D), jnp.float32),
        mesh=_sc_mesh(),
        scratch_types=[
            pltpu.VMEM((CPP, CHUNK), jnp.int32),
            pltpu.VMEM((CPP, CHUNK), jnp.int32),
            pltpu.VMEM((CHUNK, D), jnp.float32),
            pltpu.VMEM((CHUNK, D), jnp.float32),
            pltpu.VMEM_SHARED((NPAD, D), jnp.float32),
            pltpu.SemaphoreType.DMA,
            pltpu.SemaphoreType.DMA,
            pltpu.SemaphoreType.DMA,
            pltpu.SemaphoreType.DMA,
        ],
    )


def _sc_scatter(y, src_p, dst_p):
    return _sc_scatter_kernel()(y, src_p, dst_p)


def _scatter_placeholder(y, src_p, dst_p):
    s = src_p.reshape(-1)[:E]
    d = dst_p.reshape(-1)[:E]
    agg = jax.ops.segment_sum(y[s], d, num_segments=N)
    p = jnp.zeros((2, NPAD, D), jnp.float32)
    return p.at[0, :N].set(agg + y)


def _deg_placeholder(dst):
    ones = jnp.ones((E,), jnp.float32)
    deg = jax.ops.segment_sum(ones, dst, num_segments=N)
    out = jnp.zeros((2, NPAD, 16), jnp.float32)
    return out.at[0, :N, 0].set(deg)


def kernel(x, edge_index, batch, W1, b1, W2, b2, W3, b3, Wm, bm):
    src = edge_index[0]
    dst = edge_index[1]
    npad = EPAD - E
    ar = jnp.arange(npad, dtype=jnp.int32)
    pad_src = (ar * 13) % N
    pad_dst = N + (ar % (NPAD - N))
    src_p = jnp.concatenate([src, pad_src]).reshape(NW, NCHUNK, CHUNK)
    dst_p = jnp.concatenate([dst, pad_dst]).reshape(NW, NCHUNK, CHUNK)

    deg_p = _sc_deg(dst_p)

    y1, dinv = _tc_prep(x, deg_p, W1)
    p1 = _sc_scatter(y1, src_p, dst_p)
    y2 = _tc_combine(p1, dinv, b1.reshape(1, D), W2)
    p2 = _sc_scatter(y2, src_p, dst_p)
    y3 = _tc_combine(p2, dinv, b2.reshape(1, D), W3)
    p3 = _sc_scatter(y3, src_p, dst_p)
    node_emb, graph_emb, logits, probs = _tc_readout(
        p3, dinv, b3.reshape(1, D), batch.reshape(GRID, 1, RB), Wm,
        bm.reshape(1, OUT))
    return (logits, probs, node_emb, graph_emb)


# speedup vs baseline: 1.2160x; 1.2160x over previous
import functools

import jax
import jax.numpy as jnp
from jax import lax
from jax.experimental import pallas as pl
from jax.experimental.pallas import tpu as pltpu
from jax.experimental.pallas import tpu_sc as plsc

N = 10000
E = 320000
D = 128
G = 64
OUT = 10

NC = 2
NS = 16
NW = NC * NS
CHUNK = 128
NCHUNK = 80
EPW = CHUNK * NCHUNK
EPAD = NW * EPW
NPAD = 10240
RPT = NPAD // NS

NPHASE = 2
CPP = NCHUNK // NPHASE

RB = 1000
GRID = N // RB


def _prep_body(x_ref, deg_ref, w_ref, y_ref, dinv_ref):
    deg = deg_ref[0, :, 0:1] + deg_ref[1, :, 0:1] + 1.0
    dinv = lax.rsqrt(deg)
    xw = jnp.dot(x_ref[...], w_ref[...], preferred_element_type=jnp.float32)
    dinv_ref[...] = dinv
    y_ref[...] = dinv * xw


def _tc_prep(x, deg_p, w1):
    return pl.pallas_call(
        _prep_body,
        grid=(GRID,),
        in_specs=[
            pl.BlockSpec((RB, D), lambda i: (i, 0)),
            pl.BlockSpec((2, RB, D), lambda i: (0, i, 0)),
            pl.BlockSpec((D, D), lambda i: (0, 0)),
        ],
        out_specs=[
            pl.BlockSpec((RB, D), lambda i: (i, 0)),
            pl.BlockSpec((RB, 1), lambda i: (i, 0)),
        ],
        out_shape=[
            jax.ShapeDtypeStruct((N, D), jnp.float32),
            jax.ShapeDtypeStruct((N, 1), jnp.float32),
        ],
    )(x, deg_p, w1)


def _combine_body(p_ref, dinv_ref, b_ref, w_ref, ynext_ref):
    agg = p_ref[0] + p_ref[1]
    h = jnp.maximum(dinv_ref[...] * agg + b_ref[...], 0.0)
    ynext_ref[...] = dinv_ref[...] * jnp.dot(
        h, w_ref[...], preferred_element_type=jnp.float32)


def _tc_combine(p, dinv, b, wn):
    return pl.pallas_call(
        _combine_body,
        grid=(GRID,),
        in_specs=[
            pl.BlockSpec((2, RB, D), lambda i: (0, i, 0)),
            pl.BlockSpec((RB, 1), lambda i: (i, 0)),
            pl.BlockSpec((1, D), lambda i: (0, 0)),
            pl.BlockSpec((D, D), lambda i: (0, 0)),
        ],
        out_specs=pl.BlockSpec((RB, D), lambda i: (i, 0)),
        out_shape=jax.ShapeDtypeStruct((N, D), jnp.float32),
    )(p, dinv, b, wn)


def _readout_body(p_ref, dinv_ref, b_ref, batch_ref, wm_ref, bm_ref,
                  emb_ref, gemb_ref, logits_ref, probs_ref,
                  gacc, cacc):
    i = pl.program_id(0)
    agg = p_ref[0] + p_ref[1]
    h = jnp.maximum(dinv_ref[...] * agg + b_ref[...], 0.0)
    emb_ref[...] = h

    onehot = (batch_ref[0] == lax.broadcasted_iota(jnp.int32, (G, RB), 0))
    onehot = onehot.astype(jnp.float32)
    gpart = jnp.dot(onehot, h, preferred_element_type=jnp.float32)
    cpart = jnp.broadcast_to(jnp.sum(onehot, axis=1, keepdims=True), (G, D))

    @pl.when(i == 0)
    def _():
        gacc[...] = gpart
        cacc[...] = cpart

    @pl.when(i > 0)
    def _():
        gacc[...] = gacc[...] + gpart
        cacc[...] = cacc[...] + cpart

    @pl.when(i == GRID - 1)
    def _():
        ge = gacc[...] / jnp.maximum(cacc[...], 1.0)
        gemb_ref[...] = ge
        logits = jnp.dot(ge, wm_ref[...], preferred_element_type=jnp.float32)
        logits = logits + bm_ref[...]
        logits_ref[...] = logits
        m = jnp.max(logits, axis=-1, keepdims=True)
        ex = jnp.exp(logits - m)
        probs_ref[...] = ex / jnp.sum(ex, axis=-1, keepdims=True)


def _tc_readout(p, dinv, b, batch2d, wm, bm):
    return pl.pallas_call(
        _readout_body,
        grid=(GRID,),
        in_specs=[
            pl.BlockSpec((2, RB, D), lambda i: (0, i, 0)),
            pl.BlockSpec((RB, 1), lambda i: (i, 0)),
            pl.BlockSpec((1, D), lambda i: (0, 0)),
            pl.BlockSpec((1, 1, RB), lambda i: (i, 0, 0)),
            pl.BlockSpec((D, OUT), lambda i: (0, 0)),
            pl.BlockSpec((1, OUT), lambda i: (0, 0)),
        ],
        out_specs=[
            pl.BlockSpec((RB, D), lambda i: (i, 0)),
            pl.BlockSpec((G, D), lambda i: (0, 0)),
            pl.BlockSpec((G, OUT), lambda i: (0, 0)),
            pl.BlockSpec((G, OUT), lambda i: (0, 0)),
        ],
        out_shape=[
            jax.ShapeDtypeStruct((N, D), jnp.float32),
            jax.ShapeDtypeStruct((G, D), jnp.float32),
            jax.ShapeDtypeStruct((G, OUT), jnp.float32),
            jax.ShapeDtypeStruct((G, OUT), jnp.float32),
        ],
        scratch_shapes=[
            pltpu.VMEM((G, D), jnp.float32),
            pltpu.VMEM((G, D), jnp.float32),
        ],
    )(p, dinv, b, batch2d, wm, bm)


@functools.lru_cache(maxsize=None)
def _sc_mesh():
    return plsc.VectorSubcoreMesh(
        core_axis_name="c", subcore_axis_name="s",
        num_cores=NC, num_subcores=NS)


def _zero_rows(ref, nrows):
    def zb(i, _):
        for k in range(D // 16):
            ref[i, pl.ds(k * 16, 16)] = jnp.zeros((16,), jnp.float32)
        return 0
    lax.fori_loop(0, nrows, zb, 0)


def _fill_const(ref, nrows, val):
    def fb(i, _):
        for k in range(D // 16):
            ref[i, pl.ds(k * 16, 16)] = jnp.full((16,), val, jnp.float32)
        return 0
    lax.fori_loop(0, nrows, fb, 0)


def _deg_body(dst_hbm, out_hbm, dstv, ones_v, acc):
    cid = lax.axis_index("c")
    sid = lax.axis_index("s")
    wid = cid * NS + sid

    pltpu.sync_copy(dst_hbm.at[wid], dstv)

    _fill_const(ones_v, CHUNK, 0.0)
    base = sid * RPT
    for t in range(RPT // CHUNK):
        pltpu.sync_copy(ones_v, acc.at[pl.ds(base + t * CHUNK, CHUNK)])

    _fill_const(ones_v, CHUNK, 1.0)
    plsc.subcore_barrier()

    def body(j, _):
        pltpu.sync_copy(ones_v, acc.at[dstv.at[j]], add=True)
        return 0
    lax.fori_loop(0, NCHUNK, body, 0)

    plsc.subcore_barrier()
    for t in range(RPT // CHUNK):
        pltpu.sync_copy(acc.at[pl.ds(base + t * CHUNK, CHUNK)], ones_v)
        pltpu.sync_copy(ones_v, out_hbm.at[cid, pl.ds(base + t * CHUNK, CHUNK)])


@functools.lru_cache(maxsize=None)
def _sc_deg_kernel():
    return pl.kernel(
        _deg_body,
        out_type=jax.ShapeDtypeStruct((NC, NPAD, D), jnp.float32),
        mesh=_sc_mesh(),
        scratch_types=[
            pltpu.VMEM((NCHUNK, CHUNK), jnp.int32),
            pltpu.VMEM((CHUNK, D), jnp.float32),
            pltpu.VMEM_SHARED((NPAD, D), jnp.float32),
        ],
    )


def _sc_deg(dst_p):
    return _sc_deg_kernel()(dst_p)


def _scat_body(y_hbm, src_hbm, dst_hbm, out_hbm,
               srcv, dstv, rows0, rows1, acc, sem0, sem1, sems0, sems1):
    cid = lax.axis_index("c")
    sid = lax.axis_index("s")
    wid = cid * NS + sid

    @pl.when(cid == 0)
    def _():
        base = sid * RPT

        @pl.when(sid < NS - 1)
        def _():
            for t in range(RPT // CHUNK):
                pltpu.sync_copy(y_hbm.at[pl.ds(base + t * CHUNK, CHUNK)],
                                rows0)
                pltpu.sync_copy(rows0, acc.at[pl.ds(base + t * CHUNK, CHUNK)])

        @pl.when(sid == NS - 1)
        def _():
            b0 = (NS - 1) * RPT
            for t in range(3):
                pltpu.sync_copy(y_hbm.at[pl.ds(b0 + t * CHUNK, CHUNK)], rows0)
                pltpu.sync_copy(rows0, acc.at[pl.ds(b0 + t * CHUNK, CHUNK)])
            b1_ = b0 + 3 * CHUNK
            pltpu.sync_copy(y_hbm.at[pl.ds(b1_, N - b1_)],
                            rows0.at[pl.ds(0, N - b1_)])
            pltpu.sync_copy(rows0.at[pl.ds(0, N - b1_)],
                            acc.at[pl.ds(b1_, N - b1_)])
            _zero_rows(rows0, CHUNK)
            pltpu.sync_copy(rows0, acc.at[pl.ds(N, CHUNK)])
            pltpu.sync_copy(rows0.at[pl.ds(0, NPAD - N - CHUNK)],
                            acc.at[pl.ds(N + CHUNK, NPAD - N - CHUNK)])

    @pl.when(cid == 1)
    def _():
        _zero_rows(rows0, CHUNK)
        base = sid * RPT
        for t in range(RPT // CHUNK):
            pltpu.sync_copy(rows0, acc.at[pl.ds(base + t * CHUNK, CHUNK)])

    plsc.subcore_barrier()

    for ph in range(NPHASE):
        pltpu.sync_copy(src_hbm.at[wid, pl.ds(ph * CPP, CPP)], srcv)
        pltpu.sync_copy(dst_hbm.at[wid, pl.ds(ph * CPP, CPP)], dstv)

        pltpu.async_copy(y_hbm.at[srcv.at[0]], rows0, sem0)
        pltpu.async_copy(y_hbm.at[srcv.at[1]], rows1, sem1)

        def body(jj, _):
            j0 = 2 * jj
            pltpu.make_async_copy(y_hbm.at[srcv.at[j0]], rows0, sem0).wait()
            pltpu.sync_copy(rows0, acc.at[dstv.at[j0]], add=True)

            @pl.when(jj < CPP // 2 - 1)
            def _():
                pltpu.async_copy(y_hbm.at[srcv.at[j0 + 2]], rows0, sem0)

            pltpu.make_async_copy(y_hbm.at[srcv.at[j0 + 1]], rows1, sem1).wait()
            pltpu.sync_copy(rows1, acc.at[dstv.at[j0 + 1]], add=True)

            @pl.when(jj < CPP // 2 - 1)
            def _():
                pltpu.async_copy(y_hbm.at[srcv.at[j0 + 3]], rows1, sem1)
            return 0

        lax.fori_loop(0, CPP // 2, body, 0)

    plsc.subcore_barrier()
    base = sid * RPT
    for t in range(RPT // CHUNK):
        pltpu.sync_copy(acc.at[pl.ds(base + t * CHUNK, CHUNK)], rows0)
        pltpu.sync_copy(rows0, out_hbm.at[cid, pl.ds(base + t * CHUNK, CHUNK)])


@functools.lru_cache(maxsize=None)
def _sc_scatter_kernel():
    return pl.kernel(
        _scat_body,
        out_type=jax.ShapeDtypeStruct((NC, NPAD, D), jnp.float32),
        mesh=_sc_mesh(),
        scratch_types=[
            pltpu.VMEM((CPP, CHUNK), jnp.int32),
            pltpu.VMEM((CPP, CHUNK), jnp.int32),
            pltpu.VMEM((CHUNK, D), jnp.float32),
            pltpu.VMEM((CHUNK, D), jnp.float32),
            pltpu.VMEM_SHARED((NPAD, D), jnp.float32),
            pltpu.SemaphoreType.DMA,
            pltpu.SemaphoreType.DMA,
            pltpu.SemaphoreType.DMA,
            pltpu.SemaphoreType.DMA,
        ],
    )


def _sc_scatter(y, src_p, dst_p):
    return _sc_scatter_kernel()(y, src_p, dst_p)


def _scatter_placeholder(y, src_p, dst_p):
    s = src_p.reshape(-1)[:E]
    d = dst_p.reshape(-1)[:E]
    agg = jax.ops.segment_sum(y[s], d, num_segments=N)
    p = jnp.zeros((2, NPAD, D), jnp.float32)
    return p.at[0, :N].set(agg + y)


def _deg_placeholder(dst):
    ones = jnp.ones((E,), jnp.float32)
    deg = jax.ops.segment_sum(ones, dst, num_segments=N)
    out = jnp.zeros((2, NPAD, 16), jnp.float32)
    return out.at[0, :N, 0].set(deg)


def kernel(x, edge_index, batch, W1, b1, W2, b2, W3, b3, Wm, bm):
    src = edge_index[0]
    dst = edge_index[1]
    npad = EPAD - E
    ar = jnp.arange(npad, dtype=jnp.int32)
    pad_src = (ar * 13) % N
    pad_dst = N + (ar % (NPAD - N))
    src_p = jnp.concatenate([src, pad_src]).reshape(NW, NCHUNK, CHUNK)
    dst_p = jnp.concatenate([dst, pad_dst]).reshape(NW, NCHUNK, CHUNK)

    deg_p = _sc_deg(dst_p)

    y1, dinv = _tc_prep(x, deg_p, W1)
    p1 = _sc_scatter(y1, src_p, dst_p)
    y2 = _tc_combine(p1, dinv, b1.reshape(1, D), W2)
    p2 = _sc_scatter(y2, src_p, dst_p)
    y3 = _tc_combine(p2, dinv, b2.reshape(1, D), W3)
    p3 = _sc_scatter(y3, src_p, dst_p)
    node_emb, graph_emb, logits, probs = _tc_readout(
        p3, dinv, b3.reshape(1, D), batch.reshape(GRID, 1, RB), Wm,
        bm.reshape(1, OUT))
    return (logits, probs, node_emb, graph_emb)


# speedup vs baseline: 1.2322x; 1.0134x over previous
import functools

import jax
import jax.numpy as jnp
from jax import lax
from jax.experimental import pallas as pl
from jax.experimental.pallas import tpu as pltpu
from jax.experimental.pallas import tpu_sc as plsc

N = 10000
E = 320000
D = 128
G = 64
OUT = 10

NC = 2
NS = 16
NW = NC * NS
CHUNK = 128
NCHUNK = 80
EPW = CHUNK * NCHUNK
EPAD = NW * EPW
NPAD = 10240
RPT = NPAD // NS

NPHASE = 2
CPP = NCHUNK // NPHASE

RB = 1000
GRID = N // RB


def _xw_body(x_ref, w_ref, xw_ref):
    xw_ref[...] = jnp.dot(x_ref[...], w_ref[...],
                          preferred_element_type=jnp.float32)


def _tc_xw(x, w1):
    return pl.pallas_call(
        _xw_body,
        grid=(GRID,),
        in_specs=[
            pl.BlockSpec((RB, D), lambda i: (i, 0)),
            pl.BlockSpec((D, D), lambda i: (0, 0)),
        ],
        out_specs=pl.BlockSpec((RB, D), lambda i: (i, 0)),
        out_shape=jax.ShapeDtypeStruct((N, D), jnp.float32),
    )(x, w1)


def _prep_body(xw_ref, deg_ref, y_ref, dinv_ref):
    deg = deg_ref[0, :, 0:1] + deg_ref[1, :, 0:1] + 1.0
    dinv = lax.rsqrt(deg)
    dinv_ref[...] = dinv
    y_ref[...] = dinv * xw_ref[...]


def _tc_prep(xw, deg_p):
    return pl.pallas_call(
        _prep_body,
        grid=(GRID,),
        in_specs=[
            pl.BlockSpec((RB, D), lambda i: (i, 0)),
            pl.BlockSpec((2, RB, D), lambda i: (0, i, 0)),
        ],
        out_specs=[
            pl.BlockSpec((RB, D), lambda i: (i, 0)),
            pl.BlockSpec((RB, 1), lambda i: (i, 0)),
        ],
        out_shape=[
            jax.ShapeDtypeStruct((N, D), jnp.float32),
            jax.ShapeDtypeStruct((N, 1), jnp.float32),
        ],
    )(xw, deg_p)


def _combine_body(p_ref, dinv_ref, b_ref, w_ref, ynext_ref):
    agg = p_ref[0] + p_ref[1]
    h = jnp.maximum(dinv_ref[...] * agg + b_ref[...], 0.0)
    ynext_ref[...] = dinv_ref[...] * jnp.dot(
        h, w_ref[...], preferred_element_type=jnp.float32)


def _tc_combine(p, dinv, b, wn):
    return pl.pallas_call(
        _combine_body,
        grid=(GRID,),
        in_specs=[
            pl.BlockSpec((2, RB, D), lambda i: (0, i, 0)),
            pl.BlockSpec((RB, 1), lambda i: (i, 0)),
            pl.BlockSpec((1, D), lambda i: (0, 0)),
            pl.BlockSpec((D, D), lambda i: (0, 0)),
        ],
        out_specs=pl.BlockSpec((RB, D), lambda i: (i, 0)),
        out_shape=jax.ShapeDtypeStruct((N, D), jnp.float32),
    )(p, dinv, b, wn)


def _readout_body(p_ref, dinv_ref, b_ref, batch_ref, wm_ref, bm_ref,
                  emb_ref, gemb_ref, logits_ref, probs_ref,
                  gacc, cacc):
    i = pl.program_id(0)
    agg = p_ref[0] + p_ref[1]
    h = jnp.maximum(dinv_ref[...] * agg + b_ref[...], 0.0)
    emb_ref[...] = h

    onehot = (batch_ref[0] == lax.broadcasted_iota(jnp.int32, (G, RB), 0))
    onehot = onehot.astype(jnp.float32)
    gpart = jnp.dot(onehot, h, preferred_element_type=jnp.float32)
    cpart = jnp.broadcast_to(jnp.sum(onehot, axis=1, keepdims=True), (G, D))

    @pl.when(i == 0)
    def _():
        gacc[...] = gpart
        cacc[...] = cpart

    @pl.when(i > 0)
    def _():
        gacc[...] = gacc[...] + gpart
        cacc[...] = cacc[...] + cpart

    @pl.when(i == GRID - 1)
    def _():
        ge = gacc[...] / jnp.maximum(cacc[...], 1.0)
        gemb_ref[...] = ge
        logits = jnp.dot(ge, wm_ref[...], preferred_element_type=jnp.float32)
        logits = logits + bm_ref[...]
        logits_ref[...] = logits
        m = jnp.max(logits, axis=-1, keepdims=True)
        ex = jnp.exp(logits - m)
        probs_ref[...] = ex / jnp.sum(ex, axis=-1, keepdims=True)


def _tc_readout(p, dinv, b, batch2d, wm, bm):
    return pl.pallas_call(
        _readout_body,
        grid=(GRID,),
        in_specs=[
            pl.BlockSpec((2, RB, D), lambda i: (0, i, 0)),
            pl.BlockSpec((RB, 1), lambda i: (i, 0)),
            pl.BlockSpec((1, D), lambda i: (0, 0)),
            pl.BlockSpec((1, 1, RB), lambda i: (i, 0, 0)),
            pl.BlockSpec((D, OUT), lambda i: (0, 0)),
            pl.BlockSpec((1, OUT), lambda i: (0, 0)),
        ],
        out_specs=[
            pl.BlockSpec((RB, D), lambda i: (i, 0)),
            pl.BlockSpec((G, D), lambda i: (0, 0)),
            pl.BlockSpec((G, OUT), lambda i: (0, 0)),
            pl.BlockSpec((G, OUT), lambda i: (0, 0)),
        ],
        out_shape=[
            jax.ShapeDtypeStruct((N, D), jnp.float32),
            jax.ShapeDtypeStruct((G, D), jnp.float32),
            jax.ShapeDtypeStruct((G, OUT), jnp.float32),
            jax.ShapeDtypeStruct((G, OUT), jnp.float32),
        ],
        scratch_shapes=[
            pltpu.VMEM((G, D), jnp.float32),
            pltpu.VMEM((G, D), jnp.float32),
        ],
    )(p, dinv, b, batch2d, wm, bm)


@functools.lru_cache(maxsize=None)
def _sc_mesh():
    return plsc.VectorSubcoreMesh(
        core_axis_name="c", subcore_axis_name="s",
        num_cores=NC, num_subcores=NS)


def _zero_rows(ref, nrows):
    def zb(i, _):
        for k in range(D // 16):
            ref[i, pl.ds(k * 16, 16)] = jnp.zeros((16,), jnp.float32)
        return 0
    lax.fori_loop(0, nrows, zb, 0)


def _fill_const(ref, nrows, val):
    def fb(i, _):
        for k in range(D // 16):
            ref[i, pl.ds(k * 16, 16)] = jnp.full((16,), val, jnp.float32)
        return 0
    lax.fori_loop(0, nrows, fb, 0)


def _deg_body(dst_hbm, out_hbm, dstv, ones_v, acc):
    cid = lax.axis_index("c")
    sid = lax.axis_index("s")
    wid = cid * NS + sid

    pltpu.sync_copy(dst_hbm.at[wid], dstv)

    _fill_const(ones_v, CHUNK, 0.0)
    base = sid * RPT
    for t in range(RPT // CHUNK):
        pltpu.sync_copy(ones_v, acc.at[pl.ds(base + t * CHUNK, CHUNK)])

    _fill_const(ones_v, CHUNK, 1.0)
    plsc.subcore_barrier()

    def body(j, _):
        pltpu.sync_copy(ones_v, acc.at[dstv.at[j]], add=True)
        return 0
    lax.fori_loop(0, NCHUNK, body, 0)

    plsc.subcore_barrier()
    for t in range(RPT // CHUNK):
        pltpu.sync_copy(acc.at[pl.ds(base + t * CHUNK, CHUNK)], ones_v)
        pltpu.sync_copy(ones_v, out_hbm.at[cid, pl.ds(base + t * CHUNK, CHUNK)])


@functools.lru_cache(maxsize=None)
def _sc_deg_kernel():
    return pl.kernel(
        _deg_body,
        out_type=jax.ShapeDtypeStruct((NC, NPAD, D), jnp.float32),
        mesh=_sc_mesh(),
        scratch_types=[
            pltpu.VMEM((NCHUNK, CHUNK), jnp.int32),
            pltpu.VMEM((CHUNK, D), jnp.float32),
            pltpu.VMEM_SHARED((NPAD, D), jnp.float32),
        ],
    )


def _sc_deg(dst_p):
    return _sc_deg_kernel()(dst_p)


def _scat_body(y_hbm, src_hbm, dst_hbm, out_hbm,
               srcv, dstv, rows0, rows1, acc, sem0, sem1, sems0, sems1):
    cid = lax.axis_index("c")
    sid = lax.axis_index("s")
    wid = cid * NS + sid

    @pl.when(cid == 0)
    def _():
        base = sid * RPT

        @pl.when(sid < NS - 1)
        def _():
            for t in range(RPT // CHUNK):
                pltpu.sync_copy(y_hbm.at[pl.ds(base + t * CHUNK, CHUNK)],
                                rows0)
                pltpu.sync_copy(rows0, acc.at[pl.ds(base + t * CHUNK, CHUNK)])

        @pl.when(sid == NS - 1)
        def _():
            b0 = (NS - 1) * RPT
            for t in range(3):
                pltpu.sync_copy(y_hbm.at[pl.ds(b0 + t * CHUNK, CHUNK)], rows0)
                pltpu.sync_copy(rows0, acc.at[pl.ds(b0 + t * CHUNK, CHUNK)])
            b1_ = b0 + 3 * CHUNK
            pltpu.sync_copy(y_hbm.at[pl.ds(b1_, N - b1_)],
                            rows0.at[pl.ds(0, N - b1_)])
            pltpu.sync_copy(rows0.at[pl.ds(0, N - b1_)],
                            acc.at[pl.ds(b1_, N - b1_)])
            _zero_rows(rows0, CHUNK)
            pltpu.sync_copy(rows0, acc.at[pl.ds(N, CHUNK)])
            pltpu.sync_copy(rows0.at[pl.ds(0, NPAD - N - CHUNK)],
                            acc.at[pl.ds(N + CHUNK, NPAD - N - CHUNK)])

    @pl.when(cid == 1)
    def _():
        _zero_rows(rows0, CHUNK)
        base = sid * RPT
        for t in range(RPT // CHUNK):
            pltpu.sync_copy(rows0, acc.at[pl.ds(base + t * CHUNK, CHUNK)])

    plsc.subcore_barrier()

    for ph in range(NPHASE):
        pltpu.sync_copy(src_hbm.at[wid, pl.ds(ph * CPP, CPP)], srcv)
        pltpu.sync_copy(dst_hbm.at[wid, pl.ds(ph * CPP, CPP)], dstv)

        pltpu.async_copy(y_hbm.at[srcv.at[0]], rows0, sem0)
        pltpu.async_copy(y_hbm.at[srcv.at[1]], rows1, sem1)

        def body(jj, _):
            j0 = 2 * jj
            pltpu.make_async_copy(y_hbm.at[srcv.at[j0]], rows0, sem0).wait()
            pltpu.sync_copy(rows0, acc.at[dstv.at[j0]], add=True)

            @pl.when(jj < CPP // 2 - 1)
            def _():
                pltpu.async_copy(y_hbm.at[srcv.at[j0 + 2]], rows0, sem0)

            pltpu.make_async_copy(y_hbm.at[srcv.at[j0 + 1]], rows1, sem1).wait()
            pltpu.sync_copy(rows1, acc.at[dstv.at[j0 + 1]], add=True)

            @pl.when(jj < CPP // 2 - 1)
            def _():
                pltpu.async_copy(y_hbm.at[srcv.at[j0 + 3]], rows1, sem1)
            return 0

        lax.fori_loop(0, CPP // 2, body, 0)

    plsc.subcore_barrier()
    base = sid * RPT
    nw_ = RPT // CHUNK
    bufs = (rows0, rows1)
    sems = (sem0, sem1)
    pltpu.async_copy(acc.at[pl.ds(base, CHUNK)], rows0, sem0)
    for t in range(nw_):
        buf, sem = bufs[t % 2], sems[t % 2]
        pltpu.make_async_copy(acc.at[pl.ds(base + t * CHUNK, CHUNK)],
                              buf, sem).wait()
        if t < nw_ - 1:
            pltpu.async_copy(acc.at[pl.ds(base + (t + 1) * CHUNK, CHUNK)],
                             bufs[(t + 1) % 2], sems[(t + 1) % 2])
        pltpu.sync_copy(buf, out_hbm.at[cid, pl.ds(base + t * CHUNK, CHUNK)])


@functools.lru_cache(maxsize=None)
def _sc_scatter_kernel():
    return pl.kernel(
        _scat_body,
        out_type=jax.ShapeDtypeStruct((NC, NPAD, D), jnp.float32),
        mesh=_sc_mesh(),
        scratch_types=[
            pltpu.VMEM((CPP, CHUNK), jnp.int32),
            pltpu.VMEM((CPP, CHUNK), jnp.int32),
            pltpu.VMEM((CHUNK, D), jnp.float32),
            pltpu.VMEM((CHUNK, D), jnp.float32),
            pltpu.VMEM_SHARED((NPAD, D), jnp.float32),
            pltpu.SemaphoreType.DMA,
            pltpu.SemaphoreType.DMA,
            pltpu.SemaphoreType.DMA,
            pltpu.SemaphoreType.DMA,
        ],
    )


def _sc_scatter(y, src_p, dst_p):
    return _sc_scatter_kernel()(y, src_p, dst_p)


def _scatter_placeholder(y, src_p, dst_p):
    s = src_p.reshape(-1)[:E]
    d = dst_p.reshape(-1)[:E]
    agg = jax.ops.segment_sum(y[s], d, num_segments=N)
    p = jnp.zeros((2, NPAD, D), jnp.float32)
    return p.at[0, :N].set(agg + y)


def _deg_placeholder(dst):
    ones = jnp.ones((E,), jnp.float32)
    deg = jax.ops.segment_sum(ones, dst, num_segments=N)
    out = jnp.zeros((2, NPAD, 16), jnp.float32)
    return out.at[0, :N, 0].set(deg)


def kernel(x, edge_index, batch, W1, b1, W2, b2, W3, b3, Wm, bm):
    src = edge_index[0]
    dst = edge_index[1]
    npad = EPAD - E
    ar = jnp.arange(npad, dtype=jnp.int32)
    pad_src = (ar * 13) % N
    pad_dst = N + (ar % (NPAD - N))
    src_p = jnp.concatenate([src, pad_src]).reshape(NW, NCHUNK, CHUNK)
    dst_p = jnp.concatenate([dst, pad_dst]).reshape(NW, NCHUNK, CHUNK)

    xw1 = _tc_xw(x, W1)
    deg_p = _sc_deg(dst_p)

    y1, dinv = _tc_prep(xw1, deg_p)
    p1 = _sc_scatter(y1, src_p, dst_p)
    y2 = _tc_combine(p1, dinv, b1.reshape(1, D), W2)
    p2 = _sc_scatter(y2, src_p, dst_p)
    y3 = _tc_combine(p2, dinv, b2.reshape(1, D), W3)
    p3 = _sc_scatter(y3, src_p, dst_p)
    node_emb, graph_emb, logits, probs = _tc_readout(
        p3, dinv, b3.reshape(1, D), batch.reshape(GRID, 1, RB), Wm,
        bm.reshape(1, OUT))
    return (logits, probs, node_emb, graph_emb)


# speedup vs baseline: 1.2434x; 1.0091x over previous
import functools

import jax
import jax.numpy as jnp
from jax import lax
from jax.experimental import pallas as pl
from jax.experimental.pallas import tpu as pltpu
from jax.experimental.pallas import tpu_sc as plsc

N = 10000
E = 320000
D = 128
G = 64
OUT = 10

NC = 2
NS = 16
NW = NC * NS
CHUNK = 128
NCHUNK = 80
EPW = CHUNK * NCHUNK
EPAD = NW * EPW
NPAD = 10240
RPT = NPAD // NS

NPHASE = 2
CPP = NCHUNK // NPHASE

RB = 1000
GRID = N // RB


def _xw_body(x_ref, w_ref, xw_ref):
    xw_ref[...] = jnp.dot(x_ref[...], w_ref[...],
                          preferred_element_type=jnp.float32)


def _tc_xw(x, w1):
    return pl.pallas_call(
        _xw_body,
        grid=(GRID,),
        in_specs=[
            pl.BlockSpec((RB, D), lambda i: (i, 0)),
            pl.BlockSpec((D, D), lambda i: (0, 0)),
        ],
        out_specs=pl.BlockSpec((RB, D), lambda i: (i, 0)),
        out_shape=jax.ShapeDtypeStruct((N, D), jnp.float32),
    )(x, w1)


def _prep_body(xw_ref, deg_ref, y_ref, dinv_ref):
    deg = deg_ref[0, :, 0:1] + deg_ref[1, :, 0:1] + 1.0
    dinv = lax.rsqrt(deg)
    dinv_ref[...] = dinv
    y_ref[...] = dinv * xw_ref[...]


def _tc_prep(xw, deg_p):
    return pl.pallas_call(
        _prep_body,
        grid=(GRID,),
        in_specs=[
            pl.BlockSpec((RB, D), lambda i: (i, 0)),
            pl.BlockSpec((2, RB, D), lambda i: (0, i, 0)),
        ],
        out_specs=[
            pl.BlockSpec((RB, D), lambda i: (i, 0)),
            pl.BlockSpec((RB, 1), lambda i: (i, 0)),
        ],
        out_shape=[
            jax.ShapeDtypeStruct((N, D), jnp.float32),
            jax.ShapeDtypeStruct((N, 1), jnp.float32),
        ],
    )(xw, deg_p)


def _combine_body(p_ref, dinv_ref, b_ref, w_ref, ynext_ref):
    agg = p_ref[0] + p_ref[1]
    h = jnp.maximum(dinv_ref[...] * agg + b_ref[...], 0.0)
    ynext_ref[...] = dinv_ref[...] * jnp.dot(
        h, w_ref[...], preferred_element_type=jnp.float32)


def _tc_combine(p, dinv, b, wn):
    return pl.pallas_call(
        _combine_body,
        grid=(GRID,),
        in_specs=[
            pl.BlockSpec((2, RB, D), lambda i: (0, i, 0)),
            pl.BlockSpec((RB, 1), lambda i: (i, 0)),
            pl.BlockSpec((1, D), lambda i: (0, 0)),
            pl.BlockSpec((D, D), lambda i: (0, 0)),
        ],
        out_specs=pl.BlockSpec((RB, D), lambda i: (i, 0)),
        out_shape=jax.ShapeDtypeStruct((N, D), jnp.float32),
    )(p, dinv, b, wn)


def _readout_body(p_ref, dinv_ref, b_ref, batch_ref, wm_ref, bm_ref,
                  emb_ref, gemb_ref, logits_ref, probs_ref,
                  gacc, cacc):
    i = pl.program_id(0)
    agg = p_ref[0] + p_ref[1]
    h = jnp.maximum(dinv_ref[...] * agg + b_ref[...], 0.0)
    emb_ref[...] = h

    onehot = (batch_ref[0] == lax.broadcasted_iota(jnp.int32, (G, RB), 0))
    onehot = onehot.astype(jnp.float32)
    gpart = jnp.dot(onehot, h, preferred_element_type=jnp.float32)
    cpart = jnp.broadcast_to(jnp.sum(onehot, axis=1, keepdims=True), (G, D))

    @pl.when(i == 0)
    def _():
        gacc[...] = gpart
        cacc[...] = cpart

    @pl.when(i > 0)
    def _():
        gacc[...] = gacc[...] + gpart
        cacc[...] = cacc[...] + cpart

    @pl.when(i == GRID - 1)
    def _():
        ge = gacc[...] / jnp.maximum(cacc[...], 1.0)
        gemb_ref[...] = ge
        logits = jnp.dot(ge, wm_ref[...], preferred_element_type=jnp.float32)
        logits = logits + bm_ref[...]
        logits_ref[...] = logits
        m = jnp.max(logits, axis=-1, keepdims=True)
        ex = jnp.exp(logits - m)
        probs_ref[...] = ex / jnp.sum(ex, axis=-1, keepdims=True)


def _tc_readout(p, dinv, b, batch2d, wm, bm):
    return pl.pallas_call(
        _readout_body,
        grid=(GRID,),
        in_specs=[
            pl.BlockSpec((2, RB, D), lambda i: (0, i, 0)),
            pl.BlockSpec((RB, 1), lambda i: (i, 0)),
            pl.BlockSpec((1, D), lambda i: (0, 0)),
            pl.BlockSpec((1, 1, RB), lambda i: (i, 0, 0)),
            pl.BlockSpec((D, OUT), lambda i: (0, 0)),
            pl.BlockSpec((1, OUT), lambda i: (0, 0)),
        ],
        out_specs=[
            pl.BlockSpec((RB, D), lambda i: (i, 0)),
            pl.BlockSpec((G, D), lambda i: (0, 0)),
            pl.BlockSpec((G, OUT), lambda i: (0, 0)),
            pl.BlockSpec((G, OUT), lambda i: (0, 0)),
        ],
        out_shape=[
            jax.ShapeDtypeStruct((N, D), jnp.float32),
            jax.ShapeDtypeStruct((G, D), jnp.float32),
            jax.ShapeDtypeStruct((G, OUT), jnp.float32),
            jax.ShapeDtypeStruct((G, OUT), jnp.float32),
        ],
        scratch_shapes=[
            pltpu.VMEM((G, D), jnp.float32),
            pltpu.VMEM((G, D), jnp.float32),
        ],
    )(p, dinv, b, batch2d, wm, bm)


@functools.lru_cache(maxsize=None)
def _sc_mesh():
    return plsc.VectorSubcoreMesh(
        core_axis_name="c", subcore_axis_name="s",
        num_cores=NC, num_subcores=NS)


def _zero_rows(ref, nrows):
    def zb(i, _):
        for k in range(D // 16):
            ref[i, pl.ds(k * 16, 16)] = jnp.zeros((16,), jnp.float32)
        return 0
    lax.fori_loop(0, nrows, zb, 0)


def _fill_const(ref, nrows, val):
    def fb(i, _):
        for k in range(D // 16):
            ref[i, pl.ds(k * 16, 16)] = jnp.full((16,), val, jnp.float32)
        return 0
    lax.fori_loop(0, nrows, fb, 0)


def _deg_body(dst_hbm, out_hbm, dstv, ones_v, acc):
    cid = lax.axis_index("c")
    sid = lax.axis_index("s")
    wid = cid * NS + sid

    pltpu.sync_copy(dst_hbm.at[wid], dstv)

    _fill_const(ones_v, CHUNK, 0.0)
    base = sid * RPT
    for t in range(RPT // CHUNK):
        pltpu.sync_copy(ones_v, acc.at[pl.ds(base + t * CHUNK, CHUNK)])

    _fill_const(ones_v, CHUNK, 1.0)
    plsc.subcore_barrier()

    def body(j, _):
        pltpu.sync_copy(ones_v, acc.at[dstv.at[j]], add=True)
        return 0
    lax.fori_loop(0, NCHUNK, body, 0)

    plsc.subcore_barrier()
    for t in range(RPT // CHUNK):
        pltpu.sync_copy(acc.at[pl.ds(base + t * CHUNK, CHUNK)], ones_v)
        pltpu.sync_copy(ones_v, out_hbm.at[cid, pl.ds(base + t * CHUNK, CHUNK)])


@functools.lru_cache(maxsize=None)
def _sc_deg_kernel():
    return pl.kernel(
        _deg_body,
        out_type=jax.ShapeDtypeStruct((NC, NPAD, D), jnp.float32),
        mesh=_sc_mesh(),
        scratch_types=[
            pltpu.VMEM((NCHUNK, CHUNK), jnp.int32),
            pltpu.VMEM((CHUNK, D), jnp.float32),
            pltpu.VMEM_SHARED((NPAD, D), jnp.float32),
        ],
    )


def _sc_deg(dst_p):
    return _sc_deg_kernel()(dst_p)


def _scat_body(y_hbm, src_hbm, dst_hbm, out_hbm,
               srcv, dstv, rows0, rows1, acc, sem0, sem1, sems0, sems1):
    cid = lax.axis_index("c")
    sid = lax.axis_index("s")
    wid = cid * NS + sid

    ibufs = (rows0, rows1)
    isems = (sem0, sem1)

    @pl.when(cid == 0)
    def _():
        base = sid * RPT

        @pl.when(sid < NS - 1)
        def _():
            nc_ = RPT // CHUNK
            pltpu.async_copy(y_hbm.at[pl.ds(base, CHUNK)], rows0, sem0)
            for t in range(nc_):
                buf, sem = ibufs[t % 2], isems[t % 2]
                pltpu.make_async_copy(
                    y_hbm.at[pl.ds(base + t * CHUNK, CHUNK)], buf, sem).wait()
                if t < nc_ - 1:
                    pltpu.async_copy(
                        y_hbm.at[pl.ds(base + (t + 1) * CHUNK, CHUNK)],
                        ibufs[(t + 1) % 2], isems[(t + 1) % 2])
                pltpu.sync_copy(buf, acc.at[pl.ds(base + t * CHUNK, CHUNK)])

        @pl.when(sid == NS - 1)
        def _():
            b0 = (NS - 1) * RPT
            for t in range(3):
                pltpu.sync_copy(y_hbm.at[pl.ds(b0 + t * CHUNK, CHUNK)], rows0)
                pltpu.sync_copy(rows0, acc.at[pl.ds(b0 + t * CHUNK, CHUNK)])
            b1_ = b0 + 3 * CHUNK
            pltpu.sync_copy(y_hbm.at[pl.ds(b1_, N - b1_)],
                            rows0.at[pl.ds(0, N - b1_)])
            pltpu.sync_copy(rows0.at[pl.ds(0, N - b1_)],
                            acc.at[pl.ds(b1_, N - b1_)])
            _zero_rows(rows0, CHUNK)
            pltpu.sync_copy(rows0, acc.at[pl.ds(N, CHUNK)])
            pltpu.sync_copy(rows0.at[pl.ds(0, NPAD - N - CHUNK)],
                            acc.at[pl.ds(N + CHUNK, NPAD - N - CHUNK)])

    @pl.when(cid == 1)
    def _():
        _zero_rows(rows0, CHUNK)
        base = sid * RPT
        for t in range(RPT // CHUNK):
            pltpu.sync_copy(rows0, acc.at[pl.ds(base + t * CHUNK, CHUNK)])

    plsc.subcore_barrier()

    for ph in range(NPHASE):
        pltpu.sync_copy(src_hbm.at[wid, pl.ds(ph * CPP, CPP)], srcv)
        pltpu.sync_copy(dst_hbm.at[wid, pl.ds(ph * CPP, CPP)], dstv)

        pltpu.async_copy(y_hbm.at[srcv.at[0]], rows0, sem0)
        pltpu.async_copy(y_hbm.at[srcv.at[1]], rows1, sem1)

        def body(jj, _):
            j0 = 2 * jj
            pltpu.make_async_copy(y_hbm.at[srcv.at[j0]], rows0, sem0).wait()
            pltpu.sync_copy(rows0, acc.at[dstv.at[j0]], add=True)

            @pl.when(jj < CPP // 2 - 1)
            def _():
                pltpu.async_copy(y_hbm.at[srcv.at[j0 + 2]], rows0, sem0)

            pltpu.make_async_copy(y_hbm.at[srcv.at[j0 + 1]], rows1, sem1).wait()
            pltpu.sync_copy(rows1, acc.at[dstv.at[j0 + 1]], add=True)

            @pl.when(jj < CPP // 2 - 1)
            def _():
                pltpu.async_copy(y_hbm.at[srcv.at[j0 + 3]], rows1, sem1)
            return 0

        lax.fori_loop(0, CPP // 2, body, 0)

    plsc.subcore_barrier()
    base = sid * RPT
    nw_ = RPT // CHUNK
    bufs = (rows0, rows1)
    sems = (sem0, sem1)
    pltpu.async_copy(acc.at[pl.ds(base, CHUNK)], rows0, sem0)
    for t in range(nw_):
        buf, sem = bufs[t % 2], sems[t % 2]
        pltpu.make_async_copy(acc.at[pl.ds(base + t * CHUNK, CHUNK)],
                              buf, sem).wait()
        if t < nw_ - 1:
            pltpu.async_copy(acc.at[pl.ds(base + (t + 1) * CHUNK, CHUNK)],
                             bufs[(t + 1) % 2], sems[(t + 1) % 2])
        pltpu.sync_copy(buf, out_hbm.at[cid, pl.ds(base + t * CHUNK, CHUNK)])


@functools.lru_cache(maxsize=None)
def _sc_scatter_kernel():
    return pl.kernel(
        _scat_body,
        out_type=jax.ShapeDtypeStruct((NC, NPAD, D), jnp.float32),
        mesh=_sc_mesh(),
        scratch_types=[
            pltpu.VMEM((CPP, CHUNK), jnp.int32),
            pltpu.VMEM((CPP, CHUNK), jnp.int32),
            pltpu.VMEM((CHUNK, D), jnp.float32),
            pltpu.VMEM((CHUNK, D), jnp.float32),
            pltpu.VMEM_SHARED((NPAD, D), jnp.float32),
            pltpu.SemaphoreType.DMA,
            pltpu.SemaphoreType.DMA,
            pltpu.SemaphoreType.DMA,
            pltpu.SemaphoreType.DMA,
        ],
    )


def _sc_scatter(y, src_p, dst_p):
    return _sc_scatter_kernel()(y, src_p, dst_p)


def _scatter_placeholder(y, src_p, dst_p):
    s = src_p.reshape(-1)[:E]
    d = dst_p.reshape(-1)[:E]
    agg = jax.ops.segment_sum(y[s], d, num_segments=N)
    p = jnp.zeros((2, NPAD, D), jnp.float32)
    return p.at[0, :N].set(agg + y)


def _deg_placeholder(dst):
    ones = jnp.ones((E,), jnp.float32)
    deg = jax.ops.segment_sum(ones, dst, num_segments=N)
    out = jnp.zeros((2, NPAD, 16), jnp.float32)
    return out.at[0, :N, 0].set(deg)


def kernel(x, edge_index, batch, W1, b1, W2, b2, W3, b3, Wm, bm):
    src = edge_index[0]
    dst = edge_index[1]
    npad = EPAD - E
    ar = jnp.arange(npad, dtype=jnp.int32)
    pad_src = (ar * 13) % N
    pad_dst = N + (ar % (NPAD - N))
    src_p = jnp.concatenate([src, pad_src]).reshape(NW, NCHUNK, CHUNK)
    dst_p = jnp.concatenate([dst, pad_dst]).reshape(NW, NCHUNK, CHUNK)

    xw1 = _tc_xw(x, W1)
    deg_p = _sc_deg(dst_p)

    y1, dinv = _tc_prep(xw1, deg_p)
    p1 = _sc_scatter(y1, src_p, dst_p)
    y2 = _tc_combine(p1, dinv, b1.reshape(1, D), W2)
    p2 = _sc_scatter(y2, src_p, dst_p)
    y3 = _tc_combine(p2, dinv, b2.reshape(1, D), W3)
    p3 = _sc_scatter(y3, src_p, dst_p)
    node_emb, graph_emb, logits, probs = _tc_readout(
        p3, dinv, b3.reshape(1, D), batch.reshape(GRID, 1, RB), Wm,
        bm.reshape(1, OUT))
    return (logits, probs, node_emb, graph_emb)


# speedup vs baseline: 1.2441x; 1.0006x over previous
import functools

import jax
import jax.numpy as jnp
from jax import lax
from jax.experimental import pallas as pl
from jax.experimental.pallas import tpu as pltpu
from jax.experimental.pallas import tpu_sc as plsc

N = 10000
E = 320000
D = 128
G = 64
OUT = 10

NC = 2
NS = 16
NW = NC * NS
CHUNK = 128
NCHUNK = 80
EPW = CHUNK * NCHUNK
EPAD = NW * EPW
NPAD = 10240
RPT = NPAD // NS

NPHASE = 2
CPP = NCHUNK // NPHASE

RB = 1000
GRID = N // RB


def _xw_body(x_ref, w_ref, xw_ref):
    xw_ref[...] = jnp.dot(x_ref[...], w_ref[...],
                          preferred_element_type=jnp.float32)


def _tc_xw(x, w1):
    return pl.pallas_call(
        _xw_body,
        grid=(GRID,),
        in_specs=[
            pl.BlockSpec((RB, D), lambda i: (i, 0)),
            pl.BlockSpec((D, D), lambda i: (0, 0)),
        ],
        out_specs=pl.BlockSpec((RB, D), lambda i: (i, 0)),
        out_shape=jax.ShapeDtypeStruct((N, D), jnp.float32),
    )(x, w1)


def _prep_body(xw_ref, deg_ref, y_ref, dinv_ref):
    deg = deg_ref[0, :, 0:1] + deg_ref[1, :, 0:1] + 1.0
    dinv = lax.rsqrt(deg)
    dinv_ref[...] = dinv
    y_ref[...] = dinv * xw_ref[...]


def _tc_prep(xw, deg_p):
    return pl.pallas_call(
        _prep_body,
        grid=(GRID,),
        in_specs=[
            pl.BlockSpec((RB, D), lambda i: (i, 0)),
            pl.BlockSpec((2, RB, D), lambda i: (0, i, 0)),
        ],
        out_specs=[
            pl.BlockSpec((RB, D), lambda i: (i, 0)),
            pl.BlockSpec((RB, 1), lambda i: (i, 0)),
        ],
        out_shape=[
            jax.ShapeDtypeStruct((N, D), jnp.float32),
            jax.ShapeDtypeStruct((N, 1), jnp.float32),
        ],
    )(xw, deg_p)


def _combine_body(p_ref, dinv_ref, b_ref, w_ref, ynext_ref):
    agg = p_ref[0] + p_ref[1]
    h = jnp.maximum(dinv_ref[...] * agg + b_ref[...], 0.0)
    ynext_ref[...] = dinv_ref[...] * jnp.dot(
        h, w_ref[...], preferred_element_type=jnp.float32)


def _tc_combine(p, dinv, b, wn):
    return pl.pallas_call(
        _combine_body,
        grid=(GRID,),
        in_specs=[
            pl.BlockSpec((2, RB, D), lambda i: (0, i, 0)),
            pl.BlockSpec((RB, 1), lambda i: (i, 0)),
            pl.BlockSpec((1, D), lambda i: (0, 0)),
            pl.BlockSpec((D, D), lambda i: (0, 0)),
        ],
        out_specs=pl.BlockSpec((RB, D), lambda i: (i, 0)),
        out_shape=jax.ShapeDtypeStruct((N, D), jnp.float32),
    )(p, dinv, b, wn)


def _readout_body(p_ref, dinv_ref, b_ref, batch_ref, wm_ref, bm_ref,
                  emb_ref, gemb_ref, logits_ref, probs_ref,
                  gacc, cacc):
    i = pl.program_id(0)
    agg = p_ref[0] + p_ref[1]
    h = jnp.maximum(dinv_ref[...] * agg + b_ref[...], 0.0)
    emb_ref[...] = h

    onehot = (batch_ref[0] == lax.broadcasted_iota(jnp.int32, (G, RB), 0))
    onehot = onehot.astype(jnp.float32)
    gpart = jnp.dot(onehot, h, preferred_element_type=jnp.float32)
    cpart = jnp.broadcast_to(jnp.sum(onehot, axis=1, keepdims=True), (G, D))

    @pl.when(i == 0)
    def _():
        gacc[...] = gpart
        cacc[...] = cpart

    @pl.when(i > 0)
    def _():
        gacc[...] = gacc[...] + gpart
        cacc[...] = cacc[...] + cpart

    @pl.when(i == GRID - 1)
    def _():
        ge = gacc[...] / jnp.maximum(cacc[...], 1.0)
        gemb_ref[...] = ge
        logits = jnp.dot(ge, wm_ref[...], preferred_element_type=jnp.float32)
        logits = logits + bm_ref[...]
        logits_ref[...] = logits
        m = jnp.max(logits, axis=-1, keepdims=True)
        ex = jnp.exp(logits - m)
        probs_ref[...] = ex / jnp.sum(ex, axis=-1, keepdims=True)


def _tc_readout(p, dinv, b, batch2d, wm, bm):
    return pl.pallas_call(
        _readout_body,
        grid=(GRID,),
        in_specs=[
            pl.BlockSpec((2, RB, D), lambda i: (0, i, 0)),
            pl.BlockSpec((RB, 1), lambda i: (i, 0)),
            pl.BlockSpec((1, D), lambda i: (0, 0)),
            pl.BlockSpec((1, 1, RB), lambda i: (i, 0, 0)),
            pl.BlockSpec((D, OUT), lambda i: (0, 0)),
            pl.BlockSpec((1, OUT), lambda i: (0, 0)),
        ],
        out_specs=[
            pl.BlockSpec((RB, D), lambda i: (i, 0)),
            pl.BlockSpec((G, D), lambda i: (0, 0)),
            pl.BlockSpec((G, OUT), lambda i: (0, 0)),
            pl.BlockSpec((G, OUT), lambda i: (0, 0)),
        ],
        out_shape=[
            jax.ShapeDtypeStruct((N, D), jnp.float32),
            jax.ShapeDtypeStruct((G, D), jnp.float32),
            jax.ShapeDtypeStruct((G, OUT), jnp.float32),
            jax.ShapeDtypeStruct((G, OUT), jnp.float32),
        ],
        scratch_shapes=[
            pltpu.VMEM((G, D), jnp.float32),
            pltpu.VMEM((G, D), jnp.float32),
        ],
    )(p, dinv, b, batch2d, wm, bm)


@functools.lru_cache(maxsize=None)
def _sc_mesh():
    return plsc.VectorSubcoreMesh(
        core_axis_name="c", subcore_axis_name="s",
        num_cores=NC, num_subcores=NS)


def _zero_rows(ref, nrows):
    def zb(i, _):
        for k in range(D // 16):
            ref[i, pl.ds(k * 16, 16)] = jnp.zeros((16,), jnp.float32)
        return 0
    lax.fori_loop(0, nrows, zb, 0)


def _fill_const(ref, nrows, val):
    def fb(i, _):
        for k in range(D // 16):
            ref[i, pl.ds(k * 16, 16)] = jnp.full((16,), val, jnp.float32)
        return 0
    lax.fori_loop(0, nrows, fb, 0)


def _deg_body(dst_hbm, out_hbm, dstv, ones_v, acc):
    cid = lax.axis_index("c")
    sid = lax.axis_index("s")
    wid = cid * NS + sid

    pltpu.sync_copy(dst_hbm.at[wid], dstv)

    _fill_const(ones_v, CHUNK, 0.0)
    base = sid * RPT
    for t in range(RPT // CHUNK):
        pltpu.sync_copy(ones_v, acc.at[pl.ds(base + t * CHUNK, CHUNK)])

    _fill_const(ones_v, CHUNK, 1.0)
    plsc.subcore_barrier()

    def body(j, _):
        pltpu.sync_copy(ones_v, acc.at[dstv.at[j]], add=True)
        return 0
    lax.fori_loop(0, NCHUNK, body, 0)

    plsc.subcore_barrier()
    for t in range(RPT // CHUNK):
        pltpu.sync_copy(acc.at[pl.ds(base + t * CHUNK, CHUNK)], ones_v)
        pltpu.sync_copy(ones_v, out_hbm.at[cid, pl.ds(base + t * CHUNK, CHUNK)])


@functools.lru_cache(maxsize=None)
def _sc_deg_kernel():
    return pl.kernel(
        _deg_body,
        out_type=jax.ShapeDtypeStruct((NC, NPAD, D), jnp.float32),
        mesh=_sc_mesh(),
        scratch_types=[
            pltpu.VMEM((NCHUNK, CHUNK), jnp.int32),
            pltpu.VMEM((CHUNK, D), jnp.float32),
            pltpu.VMEM_SHARED((NPAD, D), jnp.float32),
        ],
    )


def _sc_deg(dst_p):
    return _sc_deg_kernel()(dst_p)


def _scat_body(y_hbm, src_hbm, dst_hbm, out_hbm,
               srcv, dstv, rows0, rows1, acc, sem0, sem1):
    cid = lax.axis_index("c")
    sid = lax.axis_index("s")
    wid = cid * NS + sid

    ibufs = (rows0, rows1)
    isems = (sem0, sem1)

    @pl.when(cid == 0)
    def _():
        base = sid * RPT

        @pl.when(sid < NS - 1)
        def _():
            nc_ = RPT // CHUNK
            pltpu.async_copy(y_hbm.at[pl.ds(base, CHUNK)], rows0, sem0)
            for t in range(nc_):
                buf, sem = ibufs[t % 2], isems[t % 2]
                pltpu.make_async_copy(
                    y_hbm.at[pl.ds(base + t * CHUNK, CHUNK)], buf, sem).wait()
                if t < nc_ - 1:
                    pltpu.async_copy(
                        y_hbm.at[pl.ds(base + (t + 1) * CHUNK, CHUNK)],
                        ibufs[(t + 1) % 2], isems[(t + 1) % 2])
                pltpu.sync_copy(buf, acc.at[pl.ds(base + t * CHUNK, CHUNK)])

        @pl.when(sid == NS - 1)
        def _():
            b0 = (NS - 1) * RPT
            for t in range(3):
                pltpu.sync_copy(y_hbm.at[pl.ds(b0 + t * CHUNK, CHUNK)], rows0)
                pltpu.sync_copy(rows0, acc.at[pl.ds(b0 + t * CHUNK, CHUNK)])
            b1_ = b0 + 3 * CHUNK
            pltpu.sync_copy(y_hbm.at[pl.ds(b1_, N - b1_)],
                            rows0.at[pl.ds(0, N - b1_)])
            pltpu.sync_copy(rows0.at[pl.ds(0, N - b1_)],
                            acc.at[pl.ds(b1_, N - b1_)])
            _zero_rows(rows0, CHUNK)
            pltpu.sync_copy(rows0, acc.at[pl.ds(N, CHUNK)])
            pltpu.sync_copy(rows0.at[pl.ds(0, NPAD - N - CHUNK)],
                            acc.at[pl.ds(N + CHUNK, NPAD - N - CHUNK)])

    @pl.when(cid == 1)
    def _():
        _zero_rows(rows0, CHUNK)
        base = sid * RPT
        for t in range(RPT // CHUNK):
            pltpu.sync_copy(rows0, acc.at[pl.ds(base + t * CHUNK, CHUNK)])

    plsc.subcore_barrier()

    for ph in range(NPHASE):
        pltpu.sync_copy(src_hbm.at[wid, pl.ds(ph * CPP, CPP)], srcv)
        pltpu.sync_copy(dst_hbm.at[wid, pl.ds(ph * CPP, CPP)], dstv)

        pltpu.async_copy(y_hbm.at[srcv.at[0]], rows0, sem0)
        pltpu.async_copy(y_hbm.at[srcv.at[1]], rows1, sem1)

        def body(jj, _):
            j0 = 2 * jj
            pltpu.make_async_copy(y_hbm.at[srcv.at[j0]], rows0, sem0).wait()
            pltpu.sync_copy(rows0, acc.at[dstv.at[j0]], add=True)

            @pl.when(jj < CPP // 2 - 1)
            def _():
                pltpu.async_copy(y_hbm.at[srcv.at[j0 + 2]], rows0, sem0)

            pltpu.make_async_copy(y_hbm.at[srcv.at[j0 + 1]], rows1, sem1).wait()
            pltpu.sync_copy(rows1, acc.at[dstv.at[j0 + 1]], add=True)

            @pl.when(jj < CPP // 2 - 1)
            def _():
                pltpu.async_copy(y_hbm.at[srcv.at[j0 + 3]], rows1, sem1)
            return 0

        lax.fori_loop(0, CPP // 2, body, 0)

    plsc.subcore_barrier()
    base = sid * RPT
    nw_ = RPT // CHUNK
    bufs = (rows0, rows1)
    sems = (sem0, sem1)
    pltpu.async_copy(acc.at[pl.ds(base, CHUNK)], rows0, sem0)
    for t in range(nw_):
        buf, sem = bufs[t % 2], sems[t % 2]
        pltpu.make_async_copy(acc.at[pl.ds(base + t * CHUNK, CHUNK)],
                              buf, sem).wait()
        if t < nw_ - 1:
            pltpu.async_copy(acc.at[pl.ds(base + (t + 1) * CHUNK, CHUNK)],
                             bufs[(t + 1) % 2], sems[(t + 1) % 2])
        pltpu.sync_copy(buf, out_hbm.at[cid, pl.ds(base + t * CHUNK, CHUNK)])


@functools.lru_cache(maxsize=None)
def _sc_scatter_kernel():
    return pl.kernel(
        _scat_body,
        out_type=jax.ShapeDtypeStruct((NC, NPAD, D), jnp.float32),
        mesh=_sc_mesh(),
        scratch_types=[
            pltpu.VMEM((CPP, CHUNK), jnp.int32),
            pltpu.VMEM((CPP, CHUNK), jnp.int32),
            pltpu.VMEM((CHUNK, D), jnp.float32),
            pltpu.VMEM((CHUNK, D), jnp.float32),
            pltpu.VMEM_SHARED((NPAD, D), jnp.float32),
            pltpu.SemaphoreType.DMA,
            pltpu.SemaphoreType.DMA,
        ],
    )


def _sc_scatter(y, src_p, dst_p):
    return _sc_scatter_kernel()(y, src_p, dst_p)


def kernel(x, edge_index, batch, W1, b1, W2, b2, W3, b3, Wm, bm):
    src = edge_index[0]
    dst = edge_index[1]
    npad = EPAD - E
    ar = jnp.arange(npad, dtype=jnp.int32)
    pad_src = (ar * 13) % N
    pad_dst = N + (ar % (NPAD - N))
    src_p = jnp.concatenate([src, pad_src]).reshape(NW, NCHUNK, CHUNK)
    dst_p = jnp.concatenate([dst, pad_dst]).reshape(NW, NCHUNK, CHUNK)

    xw1 = _tc_xw(x, W1)
    deg_p = _sc_deg(dst_p)

    y1, dinv = _tc_prep(xw1, deg_p)
    p1 = _sc_scatter(y1, src_p, dst_p)
    y2 = _tc_combine(p1, dinv, b1.reshape(1, D), W2)
    p2 = _sc_scatter(y2, src_p, dst_p)
    y3 = _tc_combine(p2, dinv, b2.reshape(1, D), W3)
    p3 = _sc_scatter(y3, src_p, dst_p)
    node_emb, graph_emb, logits, probs = _tc_readout(
        p3, dinv, b3.reshape(1, D), batch.reshape(GRID, 1, RB), Wm,
        bm.reshape(1, OUT))
    return (logits, probs, node_emb, graph_emb)
